# baseline (speedup 1.0000x reference)
import functools

import jax
import jax.numpy as jnp
from jax import lax
from jax.experimental import pallas as pl
from jax.experimental.pallas import tpu as pltpu
from jax.experimental.pallas import tpu_sc as plsc

N_NODES = 10000
D = 128
N_TYPES = 12
NC = 2
NS = 16
NW = NC * NS
CHUNK = 128
GRP = 8
N_PAD = 10240
ROWS_PER_SUB = N_PAD // NS
ZCHUNKS = ROWS_PER_SUB // CHUNK


def _build_sc_aggregate(e_pad: int):
  e_per_w = e_pad // NW
  nchunk = e_per_w // CHUNK
  mesh = plsc.VectorSubcoreMesh(
      core_axis_name="c", subcore_axis_name="s", num_cores=NC, num_subcores=NS
  )

  @functools.partial(
      pl.kernel,
      out_type=(
          jax.ShapeDtypeStruct((NC, N_PAD, D), jnp.float32),
          jax.ShapeDtypeStruct((NW, N_PAD), jnp.float32),
      ),
      mesh=mesh,
      compiler_params=pltpu.CompilerParams(needs_layout_passes=False),
      scratch_types=[
          pltpu.VMEM_SHARED((N_PAD, D), jnp.float32),
          pltpu.VMEM((GRP, 2, CHUNK), jnp.int32),
          pltpu.VMEM((ZCHUNKS, CHUNK), jnp.int32),
          pltpu.VMEM((CHUNK, D), jnp.float32),
          pltpu.VMEM((CHUNK, D), jnp.float32),
          pltpu.VMEM((N_PAD,), jnp.float32),
          pltpu.SemaphoreType.DMA,
          pltpu.SemaphoreType.DMA,
      ],
  )
  def sc_aggregate(feat_hbm, idx_hbm, ridx_hbm, acc_out, cnt_out,
                   acc_sh, idx_v, ridx_v, rows_v, rows_w, cnt_v,
                   sem_a, sem_b):
    c = lax.axis_index("c")
    s = lax.axis_index("s")
    w = c * NS + s
    base = s * ROWS_PER_SUB

    zero16 = jnp.zeros((16,), jnp.float32)
    one16 = jnp.ones((16,), jnp.float32)

    def zrow(r, carry):
      for cc in range(D // 16):
        rows_v[r, pl.ds(cc * 16, 16)] = zero16
      return carry

    lax.fori_loop(0, CHUNK, zrow, 0)

    def zcnt(r, carry):
      cnt_v[pl.ds(r * 16, 16)] = zero16
      return carry

    lax.fori_loop(0, N_PAD // 16, zcnt, 0)

    pltpu.sync_copy(ridx_hbm.at[s], ridx_v)
    for t in range(ZCHUNKS):
      pltpu.sync_copy(rows_v, acc_sh.at[ridx_v.at[t]])

    plsc.subcore_barrier()

    del rows_w, sem_a, sem_b

    def step(jg, carry):
      pltpu.sync_copy(idx_hbm.at[w, jg], idx_v)
      for g in range(GRP):
        pltpu.sync_copy(feat_hbm.at[idx_v.at[g, 0]], rows_v)
        pltpu.sync_copy(rows_v, acc_sh.at[idx_v.at[g, 1]], add=True)
        for k in range(CHUNK // 16):
          d16 = idx_v[g, 1, pl.ds(k * 16, 16)]
          plsc.addupdate_scatter(cnt_v, [d16], one16)
      return carry

    lax.fori_loop(0, nchunk // GRP, step, 0)

    plsc.subcore_barrier()

    pltpu.sync_copy(cnt_v, cnt_out.at[w])
    for t in range(ZCHUNKS):
      pltpu.sync_copy(acc_sh.at[ridx_v.at[t]], rows_v)
      pltpu.sync_copy(rows_v, acc_out.at[c, pl.ds(base + t * CHUNK, CHUNK)])

  return sc_aggregate


def _tc_body(acc_ref, cnt_ref, nt_ref, temp_ref, w_ref, b_ref, out_ref):
  ssum = acc_ref[0] + acc_ref[1]
  cnt = jnp.sum(cnt_ref[...], axis=0)
  neigh = ssum / jnp.maximum(cnt, 1.0)
  nt = nt_ref[:]
  res = temp_ref[:]
  for i in range(N_TYPES):
    oi = lax.dot_general(
        neigh, w_ref[i], (((1,), (1,)), ((), ())),
        preferred_element_type=jnp.float32,
    ) + b_ref[i][None, :]
    res = jnp.where(nt == i, oi, res)
  out_ref[:] = res


def _build_tc_apply(rblk: int):
  nblk = N_NODES // rblk
  return pl.pallas_call(
      _tc_body,
      grid=(nblk,),
      in_specs=[
          pl.BlockSpec((NC, rblk, D), lambda i: (0, i, 0)),
          pl.BlockSpec((NW, rblk, 1), lambda i: (0, i, 0)),
          pl.BlockSpec((rblk, 1), lambda i: (i, 0)),
          pl.BlockSpec((rblk, D), lambda i: (i, 0)),
          pl.BlockSpec((N_TYPES, D, D), lambda i: (0, 0, 0)),
          pl.BlockSpec((N_TYPES, D), lambda i: (0, 0)),
      ],
      out_specs=pl.BlockSpec((rblk, D), lambda i: (i, 0)),
      out_shape=jax.ShapeDtypeStruct((N_NODES, D), jnp.float32),
  )


def kernel(act_flag, feat, edge_index, ntype2, temp, gate_W, gate_b):
  e = edge_index.shape[1]
  gran = NW * CHUNK * GRP
  e_pad = ((e + gran - 1) // gran) * gran
  npad = e_pad - e
  nchunk = e_pad // (NW * CHUNK)
  src = jnp.concatenate([edge_index[0], jnp.zeros((npad,), jnp.int32)])
  dst = jnp.concatenate([edge_index[1], jnp.full((npad,), N_NODES, jnp.int32)])
  idx = jnp.stack(
      [src.reshape(NW, nchunk, CHUNK), dst.reshape(NW, nchunk, CHUNK)], axis=2
  ).reshape(NW, nchunk // GRP, GRP, 2, CHUNK)
  ridx = jnp.arange(N_PAD, dtype=jnp.int32).reshape(NS, ZCHUNKS, CHUNK)
  acc, cnt = _build_sc_aggregate(e_pad)(feat, idx, ridx)
  out = _build_tc_apply(400)(
      acc,
      cnt[:, :N_NODES].reshape(NW, N_NODES, 1),
      ntype2.reshape(N_NODES, 1),
      temp,
      gate_W,
      gate_b,
  )
  return out


# speedup vs baseline: 1.4405x; 1.4405x over previous
import functools

import jax
import jax.numpy as jnp
from jax import lax
from jax.experimental import pallas as pl
from jax.experimental.pallas import tpu as pltpu
from jax.experimental.pallas import tpu_sc as plsc

N_NODES = 10000
D = 128
N_TYPES = 12
NC = 2
NS = 16
NW = NC * NS
CHUNK = 128
GRP = 8
N_PAD = 10240
ROWS_PER_SUB = N_PAD // NS
ZCHUNKS = ROWS_PER_SUB // CHUNK


def _build_sc_aggregate(e_pad: int):
  e_per_w = e_pad // NW
  nchunk = e_per_w // CHUNK
  mesh = plsc.VectorSubcoreMesh(
      core_axis_name="c", subcore_axis_name="s", num_cores=NC, num_subcores=NS
  )

  @functools.partial(
      pl.kernel,
      out_type=(
          jax.ShapeDtypeStruct((NC, N_PAD, D), jnp.float32),
          jax.ShapeDtypeStruct((NW, N_PAD), jnp.float32),
      ),
      mesh=mesh,
      compiler_params=pltpu.CompilerParams(needs_layout_passes=False),
      scratch_types=[
          pltpu.VMEM_SHARED((N_PAD, D), jnp.float32),
          pltpu.VMEM((nchunk, 2, CHUNK), jnp.int32),
          pltpu.VMEM((ZCHUNKS, CHUNK), jnp.int32),
          pltpu.VMEM((CHUNK, D), jnp.float32),
          pltpu.VMEM((N_PAD,), jnp.float32),
      ],
  )
  def sc_aggregate(feat_hbm, idx_hbm, ridx_hbm, acc_out, cnt_out,
                   acc_sh, idx_v, ridx_v, rows_v, cnt_v):
    c = lax.axis_index("c")
    s = lax.axis_index("s")
    w = c * NS + s
    base = s * ROWS_PER_SUB

    zero16 = jnp.zeros((16,), jnp.float32)
    one16 = jnp.ones((16,), jnp.float32)

    def zrow(r, carry):
      for cc in range(D // 16):
        rows_v[r, pl.ds(cc * 16, 16)] = zero16
      return carry

    lax.fori_loop(0, CHUNK, zrow, 0)

    def zcnt(r, carry):
      cnt_v[pl.ds(r * 16, 16)] = zero16
      return carry

    lax.fori_loop(0, N_PAD // 16, zcnt, 0)

    pltpu.sync_copy(ridx_hbm.at[s], ridx_v)
    for t in range(ZCHUNKS):
      pltpu.sync_copy(rows_v, acc_sh.at[ridx_v.at[t]])

    plsc.subcore_barrier()

    pltpu.sync_copy(idx_hbm.at[w], idx_v)

    def step(j, carry):
      pltpu.sync_copy(feat_hbm.at[idx_v.at[j, 0]], rows_v)
      pltpu.sync_copy(rows_v, acc_sh.at[idx_v.at[j, 1]], add=True)
      for k in range(CHUNK // 16):
        d16 = idx_v[j, 1, pl.ds(k * 16, 16)]
        plsc.addupdate_scatter(cnt_v, [d16], one16)
      return carry

    lax.fori_loop(0, nchunk, step, 0)

    plsc.subcore_barrier()

    pltpu.sync_copy(cnt_v, cnt_out.at[w])
    for t in range(ZCHUNKS):
      pltpu.sync_copy(acc_sh.at[ridx_v.at[t]], rows_v)
      pltpu.sync_copy(rows_v, acc_out.at[c, pl.ds(base + t * CHUNK, CHUNK)])

  return sc_aggregate


def _tc_body(acc_ref, cnt_ref, nt_ref, temp_ref, w_ref, b_ref, out_ref):
  ssum = acc_ref[0] + acc_ref[1]
  cnt = jnp.sum(cnt_ref[...], axis=0)
  neigh = ssum / jnp.maximum(cnt, 1.0)
  nt = nt_ref[:]
  res = temp_ref[:]
  for i in range(N_TYPES):
    oi = lax.dot_general(
        neigh, w_ref[i], (((1,), (1,)), ((), ())),
        preferred_element_type=jnp.float32,
    ) + b_ref[i][None, :]
    res = jnp.where(nt == i, oi, res)
  out_ref[:] = res


def _build_tc_apply(rblk: int):
  nblk = N_NODES // rblk
  return pl.pallas_call(
      _tc_body,
      grid=(nblk,),
      in_specs=[
          pl.BlockSpec((NC, rblk, D), lambda i: (0, i, 0)),
          pl.BlockSpec((NW, rblk, 1), lambda i: (0, i, 0)),
          pl.BlockSpec((rblk, 1), lambda i: (i, 0)),
          pl.BlockSpec((rblk, D), lambda i: (i, 0)),
          pl.BlockSpec((N_TYPES, D, D), lambda i: (0, 0, 0)),
          pl.BlockSpec((N_TYPES, D), lambda i: (0, 0)),
      ],
      out_specs=pl.BlockSpec((rblk, D), lambda i: (i, 0)),
      out_shape=jax.ShapeDtypeStruct((N_NODES, D), jnp.float32),
  )


def kernel(act_flag, feat, edge_index, ntype2, temp, gate_W, gate_b):
  e = edge_index.shape[1]
  gran = NW * CHUNK
  e_pad = ((e + gran - 1) // gran) * gran
  npad = e_pad - e
  nchunk = e_pad // (NW * CHUNK)
  src = jnp.concatenate([edge_index[0], jnp.zeros((npad,), jnp.int32)])
  dst = jnp.concatenate([edge_index[1], jnp.full((npad,), N_NODES, jnp.int32)])
  idx = jnp.stack(
      [src.reshape(NW, nchunk, CHUNK), dst.reshape(NW, nchunk, CHUNK)], axis=2
  )
  ridx = jnp.arange(N_PAD, dtype=jnp.int32).reshape(NS, ZCHUNKS, CHUNK)
  acc, cnt = _build_sc_aggregate(e_pad)(feat, idx, ridx)
  out = _build_tc_apply(400)(
      acc,
      cnt[:, :N_NODES].reshape(NW, N_NODES, 1),
      ntype2.reshape(N_NODES, 1),
      temp,
      gate_W,
      gate_b,
  )
  return out


# speedup vs baseline: 1.4833x; 1.0297x over previous
import functools

import jax
import jax.numpy as jnp
from jax import lax
from jax.experimental import pallas as pl
from jax.experimental.pallas import tpu as pltpu
from jax.experimental.pallas import tpu_sc as plsc

N_NODES = 10000
D = 128
N_TYPES = 12
NC = 1
NS = 16
NW = NC * NS
CHUNK = 128
GRP = 8
N_PAD = 10240
ROWS_PER_SUB = N_PAD // NS
ZCHUNKS = ROWS_PER_SUB // CHUNK


def _build_sc_aggregate(e_pad: int):
  e_per_w = e_pad // NW
  nchunk = e_per_w // CHUNK
  mesh = plsc.VectorSubcoreMesh(
      core_axis_name="c", subcore_axis_name="s", num_cores=NC, num_subcores=NS
  )

  @functools.partial(
      pl.kernel,
      out_type=(
          jax.ShapeDtypeStruct((NC, N_PAD, D), jnp.float32),
          jax.ShapeDtypeStruct((NW, N_PAD), jnp.float32),
      ),
      mesh=mesh,
      compiler_params=pltpu.CompilerParams(needs_layout_passes=False),
      scratch_types=[
          pltpu.VMEM_SHARED((N_PAD, D), jnp.float32),
          pltpu.VMEM(((nchunk + 1) // 2, 2, CHUNK), jnp.int32),
          pltpu.VMEM((ZCHUNKS, CHUNK), jnp.int32),
          pltpu.VMEM((CHUNK, D), jnp.float32),
          pltpu.VMEM((N_PAD,), jnp.float32),
      ],
  )
  def sc_aggregate(feat_hbm, idx_hbm, ridx_hbm, acc_out, cnt_out,
                   acc_sh, idx_v, ridx_v, rows_v, cnt_v):
    c = lax.axis_index("c")
    s = lax.axis_index("s")
    w = c * NS + s
    base = s * ROWS_PER_SUB

    zero16 = jnp.zeros((16,), jnp.float32)
    one16 = jnp.ones((16,), jnp.float32)

    def zrow(r, carry):
      for cc in range(D // 16):
        rows_v[r, pl.ds(cc * 16, 16)] = zero16
      return carry

    lax.fori_loop(0, CHUNK, zrow, 0)

    def zcnt(r, carry):
      cnt_v[pl.ds(r * 16, 16)] = zero16
      return carry

    lax.fori_loop(0, N_PAD // 16, zcnt, 0)

    pltpu.sync_copy(ridx_hbm.at[s], ridx_v)
    for t in range(ZCHUNKS):
      pltpu.sync_copy(rows_v, acc_sh.at[ridx_v.at[t]])

    plsc.subcore_barrier()

    def step(j, carry):
      pltpu.sync_copy(feat_hbm.at[idx_v.at[j, 0]], rows_v)
      pltpu.sync_copy(rows_v, acc_sh.at[idx_v.at[j, 1]], add=True)
      for k in range(CHUNK // 16):
        d16 = idx_v[j, 1, pl.ds(k * 16, 16)]
        plsc.addupdate_scatter(cnt_v, [d16], one16)
      return carry

    half = (nchunk + 1) // 2
    for h, n_this in enumerate((half, nchunk - half)):
      pltpu.sync_copy(
          idx_hbm.at[w, pl.ds(h * half, n_this)],
          idx_v.at[pl.ds(0, n_this)],
      )
      lax.fori_loop(0, n_this, step, 0)

    plsc.subcore_barrier()

    pltpu.sync_copy(cnt_v, cnt_out.at[w])
    for t in range(ZCHUNKS):
      pltpu.sync_copy(acc_sh.at[ridx_v.at[t]], rows_v)
      pltpu.sync_copy(rows_v, acc_out.at[c, pl.ds(base + t * CHUNK, CHUNK)])

  return sc_aggregate


def _tc_body(acc_ref, cnt_ref, nt_ref, temp_ref, w_ref, b_ref, out_ref):
  ssum = jnp.sum(acc_ref[...], axis=0)
  cnt = jnp.sum(cnt_ref[...], axis=0)
  neigh = ssum / jnp.maximum(cnt, 1.0)
  nt = nt_ref[:]
  res = temp_ref[:]
  for i in range(N_TYPES):
    oi = lax.dot_general(
        neigh, w_ref[i], (((1,), (1,)), ((), ())),
        preferred_element_type=jnp.float32,
    ) + b_ref[i][None, :]
    res = jnp.where(nt == i, oi, res)
  out_ref[:] = res


def _build_tc_apply(rblk: int):
  nblk = N_NODES // rblk
  return pl.pallas_call(
      _tc_body,
      grid=(nblk,),
      in_specs=[
          pl.BlockSpec((NC, rblk, D), lambda i: (0, i, 0)),
          pl.BlockSpec((NW, rblk, 1), lambda i: (0, i, 0)),
          pl.BlockSpec((rblk, 1), lambda i: (i, 0)),
          pl.BlockSpec((rblk, D), lambda i: (i, 0)),
          pl.BlockSpec((N_TYPES, D, D), lambda i: (0, 0, 0)),
          pl.BlockSpec((N_TYPES, D), lambda i: (0, 0)),
      ],
      out_specs=pl.BlockSpec((rblk, D), lambda i: (i, 0)),
      out_shape=jax.ShapeDtypeStruct((N_NODES, D), jnp.float32),
  )


def kernel(act_flag, feat, edge_index, ntype2, temp, gate_W, gate_b):
  e = edge_index.shape[1]
  gran = NW * CHUNK
  e_pad = ((e + gran - 1) // gran) * gran
  npad = e_pad - e
  nchunk = e_pad // (NW * CHUNK)
  src = jnp.concatenate([edge_index[0], jnp.zeros((npad,), jnp.int32)])
  dst = jnp.concatenate([edge_index[1], jnp.full((npad,), N_NODES, jnp.int32)])
  idx = jnp.stack(
      [src.reshape(NW, nchunk, CHUNK), dst.reshape(NW, nchunk, CHUNK)], axis=2
  )
  ridx = jnp.arange(N_PAD, dtype=jnp.int32).reshape(NS, ZCHUNKS, CHUNK)
  acc, cnt = _build_sc_aggregate(e_pad)(feat, idx, ridx)
  out = _build_tc_apply(400)(
      acc,
      cnt[:, :N_NODES].reshape(NW, N_NODES, 1),
      ntype2.reshape(N_NODES, 1),
      temp,
      gate_W,
      gate_b,
  )
  return out
